# baseline (speedup 1.0000x reference)
import jax
import jax.numpy as jnp
from jax.experimental import pallas as pl

NUM_CLASS = 19
NUM_SUP = 2048
NUM_SMALL = 8192


def _per_image(x, xw, trg, m, vw, sup_w, small_w, ids_s):
    C, Ps = x.shape
    _, Pw = xw.shape
    x = x.T
    xw = xw.T
    mf = m.astype(jnp.float32)

    mx = jnp.max(x, axis=1)
    se = jnp.sum(jnp.exp(x - mx[:, None]), axis=1)
    t = mx + jnp.log(se)
    u = x - t[:, None]
    A = jax.ops.segment_sum(x * mf[:, None], ids_s, num_segments=NUM_SMALL)
    B = jax.ops.segment_sum(t * mf, ids_s, num_segments=NUM_SMALL)
    size = jax.ops.segment_sum(mf, ids_s, num_segments=NUM_SMALL)
    um = jnp.where(m[:, None], u, -jnp.inf)
    M = jax.ops.segment_max(um, ids_s, num_segments=NUM_SMALL)

    mxw = jnp.max(xw, axis=1)
    sew = jnp.sum(jnp.exp(xw - mxw[:, None]), axis=1)
    tw = mxw + jnp.log(sew)
    uw = xw - tw[:, None]
    uwm = jnp.where(vw[:, None], uw, -jnp.inf)
    segmax_w = jax.ops.segment_max(uwm, sup_w, num_segments=NUM_SUP)
    is_max = vw[:, None] & (uw == segmax_w[sup_w])
    idxc = jnp.where(is_max, jnp.arange(Pw)[:, None], Pw)
    amax = jax.ops.segment_min(idxc, sup_w, num_segments=NUM_SUP)
    amax = jnp.minimum(amax, Pw)
    sup_valid = amax[:, 0] < Pw
    p_sel = jnp.clip(amax, 0, Pw - 1)
    sel = small_w[p_sel]

    pair = sup_valid[:, None] & (trg > 0)
    cls = jnp.arange(C)[None, :]
    val = B[sel] - A[sel, cls]
    w = jnp.exp(M[sel, cls])
    loss_i = jnp.sum(jnp.where(pair, w * val, 0.0))
    nv_i = jnp.sum(jnp.where(pair, size[sel], 0.0))
    return loss_i, nv_i


def _div_kernel(a_ref, b_ref, o_ref):
    o_ref[...] = a_ref[...] / b_ref[...]


def kernel(inputs, inputs_weak, targets, spmasks, spmasks_weak,
           superpixels, superpixels_weak, superpixel_smalls, spx_smalls_weak):
    N, C, H, W = inputs.shape
    _, _, Hw, Ww = inputs_weak.shape
    Ps, Pw = H * W, Hw * Ww
    x = inputs.reshape(N, C, Ps)
    xw = inputs_weak.reshape(N, C, Pw)
    trg = targets[..., :-1].astype(jnp.float32)
    m = spmasks.reshape(N, Ps)
    vw = spmasks_weak.reshape(N, Pw)
    sup_w = superpixels_weak.reshape(N, Pw)
    small_w = spx_smalls_weak.reshape(N, Pw)
    ids_s = superpixel_smalls.reshape(N, Ps)
    loss_i, nv_i = jax.vmap(_per_image)(x, xw, trg, m, vw, sup_w, small_w, ids_s)
    loss = jnp.sum(loss_i)
    nv = 1.0 + jnp.sum(nv_i)
    return pl.pallas_call(
        _div_kernel,
        out_shape=jax.ShapeDtypeStruct((1,), jnp.float32),
    )(loss.reshape(1), nv.reshape(1))[0]


# speedup vs baseline: 1.0552x; 1.0552x over previous
import jax
import jax.numpy as jnp
from jax import lax
from jax.experimental import pallas as pl

NUM_CLASS = 19
NUM_SUP = 2048
NUM_SMALL = 8192

f32 = jnp.float32
i32 = jnp.int32


def _rows_body(x_ref, m_ref, t_ref, u_ref):
    x = x_ref[0]
    m = m_ref[0]
    mx = jnp.max(x, axis=0, keepdims=True)
    t = mx + jnp.log(jnp.sum(jnp.exp(x - mx), axis=0, keepdims=True))
    t_ref[0] = t * m
    u_ref[0] = x - t


def _make_rows(x, mf, bw):
    n, c, p = x.shape
    grid = (n, p // bw)
    return pl.pallas_call(
        _rows_body,
        grid=grid,
        in_specs=[
            pl.BlockSpec((1, c, bw), lambda i, j: (i, 0, j)),
            pl.BlockSpec((1, 1, bw), lambda i, j: (i, 0, j)),
        ],
        out_specs=[
            pl.BlockSpec((1, 1, bw), lambda i, j: (i, 0, j)),
            pl.BlockSpec((1, c, bw), lambda i, j: (i, 0, j)),
        ],
        out_shape=[
            jax.ShapeDtypeStruct((n, 1, p), f32),
            jax.ShapeDtypeStruct((n, c, p), f32),
        ],
    )(x, mf)


def _per_image(u, tm, uw, trg, m, vw, sup_w, small_w, ids_s):
    cdim, ps = u.shape
    _, pw = uw.shape
    u = u.T
    uw = uw.T
    mf = m.astype(f32)

    a_tab = jax.ops.segment_sum(u * mf[:, None], ids_s, num_segments=NUM_SMALL)
    size = jax.ops.segment_sum(mf, ids_s, num_segments=NUM_SMALL)
    um = jnp.where(m[:, None], u, -jnp.inf)
    m_tab = jax.ops.segment_max(um, ids_s, num_segments=NUM_SMALL)
    small_sum = -a_tab

    uwm = jnp.where(vw[:, None], uw, -jnp.inf)
    segmax_w = jax.ops.segment_max(uwm, sup_w, num_segments=NUM_SUP)
    is_max = vw[:, None] & (uw == segmax_w[sup_w])
    idxc = jnp.where(is_max, jnp.arange(pw)[:, None], pw)
    amax = jax.ops.segment_min(idxc, sup_w, num_segments=NUM_SUP)
    amax = jnp.minimum(amax, pw)
    sup_valid = amax[:, 0] < pw
    p_sel = jnp.clip(amax, 0, pw - 1)
    sel = small_w[p_sel]

    pair = sup_valid[:, None] & (trg > 0)
    cls = jnp.arange(cdim)[None, :]
    val = small_sum[sel, cls]
    w = jnp.exp(m_tab[sel, cls])
    loss_i = jnp.sum(jnp.where(pair, w * val, 0.0))
    nv_i = jnp.sum(jnp.where(pair, size[sel], 0.0))
    return loss_i, nv_i


def _final_body(l_ref, n_ref, o_ref):
    loss = jnp.sum(l_ref[...])
    nv = jnp.sum(n_ref[...])
    o_ref[...] = jnp.full((1, 1), loss / (1.0 + nv), f32)


def kernel(inputs, inputs_weak, targets, spmasks, spmasks_weak,
           superpixels, superpixels_weak, superpixel_smalls, spx_smalls_weak):
    n, c, h, w = inputs.shape
    _, _, hw, ww = inputs_weak.shape
    ps, pw = h * w, hw * ww
    x = inputs.reshape(n, c, ps)
    xw = inputs_weak.reshape(n, c, pw)
    mf = spmasks.reshape(n, 1, ps).astype(f32)
    mwf = spmasks_weak.reshape(n, 1, pw).astype(f32)
    trg = targets[..., :-1].astype(f32)
    m = spmasks.reshape(n, ps)
    vw = spmasks_weak.reshape(n, pw)
    sup_w = superpixels_weak.reshape(n, pw).astype(i32)
    small_w = spx_smalls_weak.reshape(n, pw).astype(i32)
    ids_s = superpixel_smalls.reshape(n, ps).astype(i32)

    tm_s, u_s = _make_rows(x, mf, bw=4096)
    _, u_w = _make_rows(xw, mwf, bw=4096)

    loss_i, nv_i = jax.vmap(_per_image)(
        u_s, tm_s.reshape(n, ps), u_w, trg, m, vw, sup_w, small_w, ids_s)

    out = pl.pallas_call(
        _final_body,
        out_shape=jax.ShapeDtypeStruct((1, 1), f32),
    )(loss_i.reshape(1, n), nv_i.reshape(1, n))
    return out[0, 0]
